# baseline (speedup 1.0000x reference)
import functools

import jax
import jax.numpy as jnp
from jax import lax
from jax.experimental import pallas as pl
from jax.experimental.pallas import tpu as pltpu
from jax.experimental.pallas import tpu_sc as plsc

NC = 2
NS = 16
LANES = 16
CHUNK = 128


@functools.lru_cache(maxsize=None)
def _build(B, H, W, C, N):
    npts = B * N
    nworkers = NC * NS
    assert npts % nworkers == 0
    ppw = npts // nworkers
    assert ppw % CHUNK == 0
    nchunks = ppw // CHUNK
    assert N % ppw == 0
    assert C % LANES == 0
    cgroups = C // LANES
    hw = H * W

    mesh = plsc.VectorSubcoreMesh(core_axis_name="c", subcore_axis_name="s")

    def body(feat, xs, ys, mask_o, val_o,
             x_v, y_v, i00, i01, i10, i11, w00, w01, w10, w11, mk,
             r00, r01, r10, r11, outv, sem):
        wid = lax.axis_index("s") * NC + lax.axis_index("c")
        base = wid * ppw
        bhw = (base // N) * hw

        def chunk_body(ci, _):
            pbase = base + ci * CHUNK
            pltpu.sync_copy(xs.at[pl.ds(pbase, CHUNK)], x_v)
            pltpu.sync_copy(ys.at[pl.ds(pbase, CHUNK)], y_v)
            for g in range(CHUNK // LANES):
                sl = pl.ds(g * LANES, LANES)
                xv = x_v[sl]
                yv = y_v[sl]
                xi = xv.astype(jnp.int32)
                yi = yv.astype(jnp.int32)
                fx1 = xv - xi.astype(jnp.float32)
                fy1 = yv - yi.astype(jnp.float32)
                fx0 = 1.0 - fx1
                fy0 = 1.0 - fy1
                inb = ((xv >= 0.0) & (xv <= W - 1.0)
                       & (yv >= 0.0) & (yv <= H - 1.0))
                mf = jnp.where(inb, jnp.float32(1.0), jnp.float32(0.0))
                x0 = jnp.clip(xi, 0, W - 1)
                x1 = jnp.clip(xi + 1, 0, W - 1)
                y0 = jnp.clip(yi, 0, H - 1)
                y1 = jnp.clip(yi + 1, 0, H - 1)
                r0 = y0 * W + bhw
                r1 = y1 * W + bhw
                i00[sl] = r0 + x0
                i01[sl] = r0 + x1
                i10[sl] = r1 + x0
                i11[sl] = r1 + x1
                a0 = fy0 * mf
                a1 = fy1 * mf
                w00[sl] = a0 * fx0
                w01[sl] = a0 * fx1
                w10[sl] = a1 * fx0
                w11[sl] = a1 * fx1
                mk[sl] = mf
            c0 = pltpu.async_copy(feat.at[i00], r00, sem)
            c1 = pltpu.async_copy(feat.at[i01], r01, sem)
            c2 = pltpu.async_copy(feat.at[i10], r10, sem)
            c3 = pltpu.async_copy(feat.at[i11], r11, sem)
            c0.wait()
            c1.wait()
            c2.wait()
            c3.wait()

            def grp_body(gp, _):
                pb = gp * LANES
                wv00 = w00[pl.ds(pb, LANES)]
                wv01 = w01[pl.ds(pb, LANES)]
                wv10 = w10[pl.ds(pb, LANES)]
                wv11 = w11[pl.ds(pb, LANES)]
                for j in range(LANES):
                    p = pb + j
                    wa = wv00[j]
                    wb = wv01[j]
                    wc = wv10[j]
                    wd = wv11[j]
                    for g in range(cgroups):
                        s2 = pl.ds(g * LANES, LANES)
                        outv[p, s2] = (r00[p, s2] * wa + r01[p, s2] * wb
                                       + r10[p, s2] * wc + r11[p, s2] * wd)
                return 0

            lax.fori_loop(0, CHUNK // LANES, grp_body, 0)
            pltpu.sync_copy(mk, mask_o.at[pl.ds(pbase, CHUNK)])
            pltpu.sync_copy(outv, val_o.at[pl.ds(pbase, CHUNK)])
            return 0

        lax.fori_loop(0, nchunks, chunk_body, 0)

    return pl.kernel(
        body,
        out_type=(
            jax.ShapeDtypeStruct((npts,), jnp.float32),
            jax.ShapeDtypeStruct((npts, C), jnp.float32),
        ),
        mesh=mesh,
        compiler_params=pltpu.CompilerParams(use_tc_tiling_on_sc=False),
        scratch_types=[
            pltpu.VMEM((CHUNK,), jnp.float32),
            pltpu.VMEM((CHUNK,), jnp.float32),
            pltpu.VMEM((CHUNK,), jnp.int32),
            pltpu.VMEM((CHUNK,), jnp.int32),
            pltpu.VMEM((CHUNK,), jnp.int32),
            pltpu.VMEM((CHUNK,), jnp.int32),
            pltpu.VMEM((CHUNK,), jnp.float32),
            pltpu.VMEM((CHUNK,), jnp.float32),
            pltpu.VMEM((CHUNK,), jnp.float32),
            pltpu.VMEM((CHUNK,), jnp.float32),
            pltpu.VMEM((CHUNK,), jnp.float32),
            pltpu.VMEM((CHUNK, C), jnp.float32),
            pltpu.VMEM((CHUNK, C), jnp.float32),
            pltpu.VMEM((CHUNK, C), jnp.float32),
            pltpu.VMEM((CHUNK, C), jnp.float32),
            pltpu.VMEM((CHUNK, C), jnp.float32),
            pltpu.SemaphoreType.DMA,
        ],
    )


def kernel(feature, coordinate):
    B, H, W, C = feature.shape
    N = coordinate.shape[1]
    feat = feature.reshape(B * H * W, C)
    xs = coordinate[..., 0].reshape(-1)
    ys = coordinate[..., 1].reshape(-1)
    mask_f, val_f = _build(B, H, W, C, N)(feat, xs, ys)
    return mask_f.reshape(B, N), val_f.reshape(B, N, C)
